# Initial kernel scaffold; baseline (speedup 1.0000x reference)
import jax
import jax.numpy as jnp
from jax.experimental import pallas as pl


def kernel(h_click, h_buy, pos_edge_click, neg_edge_click, pos_edge_buy, neg_edge_buy):
    raise NotImplementedError("write your pallas kernel here")


# speedup vs baseline: 1.8283x; 1.8283x over previous
import functools

import jax
import jax.numpy as jnp
from jax import lax
from jax.experimental import pallas as pl
from jax.experimental.pallas import tpu as pltpu
from jax.experimental.pallas import tpu_sc as plsc

_N_NODES = 100000
_D = 128
_E_POS = 100000
_NEG = 5
_E_NEG = _E_POS * _NEG
_E_TOT = _E_POS + _E_NEG
_CHUNK = 128
_NW = 32
_NC = 2
_CPT = 148
_EPT = _CPT * _CHUNK
_E_PAD = _NW * _EPT
_NCHUNK = _NW * _CPT


def _sc_logits(h_click, h_buy, iu_c, iv_c, iu_b, iv_b):
    mesh = plsc.VectorSubcoreMesh(core_axis_name="c", subcore_axis_name="s")

    @functools.partial(
        pl.kernel,
        out_type=(jax.ShapeDtypeStruct((_NW, _EPT), jnp.float32),
                  jax.ShapeDtypeStruct((_NW, _EPT), jnp.float32)),
        mesh=mesh,
        scratch_types=[
            pltpu.VMEM((_CPT, _CHUNK), jnp.int32),
            pltpu.VMEM((_CPT, _CHUNK), jnp.int32),
            pltpu.VMEM((_CHUNK, _D), jnp.float32),
            pltpu.VMEM((_CHUNK, _D), jnp.float32),
            pltpu.VMEM((_CHUNK, _D), jnp.float32),
            pltpu.VMEM((_CHUNK, _D), jnp.float32),
            pltpu.VMEM((_EPT,), jnp.float32),
            pltpu.VMEM((16, 32), jnp.float32),
            pltpu.VMEM((31,), jnp.float32),
            pltpu.SemaphoreType.DMA,
            pltpu.SemaphoreType.DMA,
        ],
    )
    def k(h_c, h_b, iuc, ivc, iub, ivb, out_c, out_b,
          idx_u, idx_v, ru0, rv0, ru1, rv1, logits, fbuf, gbuf, sem0, sem1):
        wid = lax.axis_index("s") * _NC + lax.axis_index("c")

        def compute_chunk(ru, rv, c):
            def group(g, carry):
                for e in range(16):
                    edge = g * 16 + e
                    acc = ru[edge, pl.ds(0, 16)] * rv[edge, pl.ds(0, 16)]
                    for kk in range(1, 8):
                        acc = acc + (ru[edge, pl.ds(kk * 16, 16)]
                                     * rv[edge, pl.ds(kk * 16, 16)])
                    fbuf[e, pl.ds(0, 16)] = acc
                    t = acc + fbuf[e, pl.ds(8, 16)]
                    fbuf[e, pl.ds(0, 16)] = t
                    t = t + fbuf[e, pl.ds(4, 16)]
                    fbuf[e, pl.ds(0, 16)] = t
                    t = t + fbuf[e, pl.ds(2, 16)]
                    fbuf[e, pl.ds(0, 16)] = t
                    t = t + fbuf[e, pl.ds(1, 16)]
                    gbuf[pl.ds(e, 16)] = t
                logits[pl.ds(c * _CHUNK + g * 16, 16)] = gbuf[pl.ds(0, 16)]
                return carry
            lax.fori_loop(0, _CHUNK // 16, group, 0)

        def run_graph(h, iu, iv, out):
            pltpu.sync_copy(iu.at[wid], idx_u)
            pltpu.sync_copy(iv.at[wid], idx_v)

            def issue(c, ru, rv, sem):
                pltpu.async_copy(h.at[idx_u.at[c]], ru, sem)
                pltpu.async_copy(h.at[idx_v.at[c]], rv, sem)

            def drain(ru, rv, sem):
                pltpu.make_async_copy(h.at[pl.ds(0, _CHUNK)], ru, sem).wait()
                pltpu.make_async_copy(h.at[pl.ds(0, _CHUNK)], rv, sem).wait()

            issue(0, ru0, rv0, sem0)

            def step(j, carry):
                c0 = j * 2
                issue(c0 + 1, ru1, rv1, sem1)
                drain(ru0, rv0, sem0)
                compute_chunk(ru0, rv0, c0)

                @pl.when(j + 1 < _CPT // 2)
                def _issue_next():
                    issue(c0 + 2, ru0, rv0, sem0)

                drain(ru1, rv1, sem1)
                compute_chunk(ru1, rv1, c0 + 1)
                return carry
            lax.fori_loop(0, _CPT // 2, step, 0)
            pltpu.sync_copy(logits, out.at[wid])

        run_graph(h_c, iuc, ivc, out_c)
        run_graph(h_b, iub, ivb, out_b)

    return k(h_click, h_buy, iu_c, iv_c, iu_b, iv_b)


def _log_sigmoid(x):
    return jnp.minimum(x, 0.0) - jnp.log1p(jnp.exp(-jnp.abs(x)))


def _tc_reduce(pos_c, neg_c, pos_b, neg_b):
    def body(pc, ngc, pb, ngb, loss_ref, mrr_ref):
        def one(p_ref, ng_ref):
            p = p_ref[...]
            ng = ng_ref[...]
            sp = jnp.sum(_log_sigmoid(p))
            sn = jnp.sum(_log_sigmoid(-ng))
            cnt = jnp.sum(jnp.where(ng >= p, 1.0, 0.0), axis=0)
            srr = jnp.sum(1.0 / (1.0 + cnt))
            loss_g = -(sp / _E_POS + sn / _E_NEG)
            return loss_g, srr / _E_POS
        l1, m1 = one(pc, ngc)
        l2, m2 = one(pb, ngb)
        loss_ref[0, 0] = (l1 + l2) * 0.5
        mrr_ref[0, 0] = (m1 + m2) * 0.5

    out = pl.pallas_call(
        body,
        out_shape=(jax.ShapeDtypeStruct((1, 1), jnp.float32),
                   jax.ShapeDtypeStruct((1, 1), jnp.float32)),
        out_specs=(pl.BlockSpec(memory_space=pltpu.SMEM),
                   pl.BlockSpec(memory_space=pltpu.SMEM)),
    )(pos_c, neg_c, pos_b, neg_b)
    return out[0][0, 0], out[1][0, 0]


def _edge_lists(pe, ne):
    pe = pe.astype(jnp.int32)
    ne = ne.astype(jnp.int32)
    nt = ne.reshape(2, _E_POS, _NEG).transpose(0, 2, 1).reshape(2, _E_NEG)
    pad = jnp.zeros((_E_PAD - _E_TOT,), jnp.int32)
    u = jnp.concatenate([pe[0], nt[0], pad]).reshape(_NW, _CPT, _CHUNK)
    v = jnp.concatenate([pe[1], nt[1], pad]).reshape(_NW, _CPT, _CHUNK)
    return u, v


def kernel(h_click, h_buy, pos_edge_click, neg_edge_click,
           pos_edge_buy, neg_edge_buy):
    iu_c, iv_c = _edge_lists(pos_edge_click, neg_edge_click)
    iu_b, iv_b = _edge_lists(pos_edge_buy, neg_edge_buy)
    lg_c, lg_b = _sc_logits(h_click, h_buy, iu_c, iv_c, iu_b, iv_b)
    lg_c = lg_c.reshape(_E_PAD)
    lg_b = lg_b.reshape(_E_PAD)
    pos_c = lg_c[:_E_POS].reshape(1, _E_POS)
    neg_c = lg_c[_E_POS:_E_TOT].reshape(_NEG, _E_POS)
    pos_b = lg_b[:_E_POS].reshape(1, _E_POS)
    neg_b = lg_b[_E_POS:_E_TOT].reshape(_NEG, _E_POS)
    loss, mrr = _tc_reduce(pos_c, neg_c, pos_b, neg_b)
    return loss, mrr
